# Initial kernel scaffold; baseline (speedup 1.0000x reference)
import jax
import jax.numpy as jnp
from jax.experimental import pallas as pl


def kernel(x, table):
    raise NotImplementedError("write your pallas kernel here")


# speedup vs baseline: 1.0823x; 1.0823x over previous
import functools

import jax
import jax.numpy as jnp
from jax import lax
from jax.experimental import pallas as pl
from jax.experimental.pallas import tpu as pltpu
from jax.experimental.pallas import tpu_sc as plsc

_EMBED_DIM = 32
_NUM_WORKERS = 32
_CHUNK = 1024


def _make_gather(num_rows: int):
    assert num_rows % (_NUM_WORKERS * _CHUNK) == 0
    rows_per_worker = num_rows // _NUM_WORKERS
    num_chunks = rows_per_worker // _CHUNK

    mesh = plsc.VectorSubcoreMesh(core_axis_name="c", subcore_axis_name="s")

    @functools.partial(
        pl.kernel,
        mesh=mesh,
        out_type=jax.ShapeDtypeStruct((num_rows, _EMBED_DIM), jnp.float32),
        compiler_params=pltpu.CompilerParams(use_tc_tiling_on_sc=False),
        scratch_types=[
            pltpu.VMEM((_CHUNK,), jnp.int32),
            pltpu.VMEM((_CHUNK, _EMBED_DIM), jnp.float32),
            pltpu.VMEM((16,), jnp.int32),
            pltpu.VMEM((16, _EMBED_DIM), jnp.float32),
            pltpu.SemaphoreType.DMA,
        ],
    )
    def gather_kernel(x_hbm, table_hbm, out_hbm, idx_v, rows_v, ridx_v,
                      rrows_v, sem):
        wid = lax.axis_index("s") * 2 + lax.axis_index("c")
        base = wid * rows_per_worker

        ridx_v[...] = jnp.minimum(lax.iota(jnp.int32, 16) + 1, 3)
        pltpu.async_copy(table_hbm.at[ridx_v], rrows_v, sem).wait()
        rep_lo = (rrows_v[0, pl.ds(0, 16)] + rrows_v[1, pl.ds(0, 16)]
                  + rrows_v[2, pl.ds(0, 16)])
        rep_hi = (rrows_v[0, pl.ds(16, 16)] + rrows_v[1, pl.ds(16, 16)]
                  + rrows_v[2, pl.ds(16, 16)])

        def chunk_body(i, _):
            off = base + i * _CHUNK
            pltpu.sync_copy(x_hbm.at[pl.ds(off, _CHUNK)], idx_v)
            pltpu.async_copy(table_hbm.at[idx_v], rows_v, sem).wait()

            def vmin_body(g, mcarry):
                return jnp.minimum(mcarry, idx_v[pl.ds(g * 16, 16)])

            m = lax.fori_loop(1, _CHUNK // 16, vmin_body,
                              idx_v[pl.ds(0, 16)])
            s = m[0]
            for j in range(1, 16):
                s = jnp.minimum(s, m[j])

            @pl.when(s == 0)
            def _fix():
                def group_body(g, _):
                    iv = idx_v[pl.ds(g * 16, 16)]
                    for j in range(16):

                        @pl.when(iv[j] == 0)
                        def _wr(j=j):
                            row = g * 16 + j
                            rows_v[row, pl.ds(0, 16)] = rep_lo
                            rows_v[row, pl.ds(16, 16)] = rep_hi

                    return 0

                lax.fori_loop(0, _CHUNK // 16, group_body, 0)
            pltpu.sync_copy(rows_v, out_hbm.at[pl.ds(off, _CHUNK)])
            return 0

        lax.fori_loop(0, num_chunks, chunk_body, 0)

    return gather_kernel


def kernel(x, table):
    xf = x.reshape(-1).astype(jnp.int32)
    out = _make_gather(xf.shape[0])(xf, table)
    return out.reshape(x.shape + (_EMBED_DIM,))


# speedup vs baseline: 1.1049x; 1.0209x over previous
import functools

import jax
import jax.numpy as jnp
from jax import lax
from jax.experimental import pallas as pl
from jax.experimental.pallas import tpu as pltpu
from jax.experimental.pallas import tpu_sc as plsc

_EMBED_DIM = 32
_NUM_WORKERS = 32
_CHUNK = 1600


def _make_gather(num_rows: int):
    assert num_rows % (_NUM_WORKERS * _CHUNK) == 0
    rows_per_worker = num_rows // _NUM_WORKERS
    num_chunks = rows_per_worker // _CHUNK
    assert num_chunks % 2 == 0 and num_chunks >= 4

    mesh = plsc.VectorSubcoreMesh(core_axis_name="c", subcore_axis_name="s")

    @functools.partial(
        pl.kernel,
        mesh=mesh,
        out_type=jax.ShapeDtypeStruct((num_rows, _EMBED_DIM), jnp.float32),
        compiler_params=pltpu.CompilerParams(use_tc_tiling_on_sc=False),
        scratch_types=[
            pltpu.VMEM((_CHUNK,), jnp.int32),
            pltpu.VMEM((_CHUNK,), jnp.int32),
            pltpu.VMEM((_CHUNK, _EMBED_DIM), jnp.float32),
            pltpu.VMEM((_CHUNK, _EMBED_DIM), jnp.float32),
            pltpu.VMEM((16,), jnp.int32),
            pltpu.VMEM((16, _EMBED_DIM), jnp.float32),
            pltpu.SemaphoreType.DMA,
            pltpu.SemaphoreType.DMA,
            pltpu.SemaphoreType.DMA,
            pltpu.SemaphoreType.DMA,
        ],
    )
    def gather_kernel(x_hbm, table_hbm, out_hbm, idx0, idx1, rows0, rows1,
                      ridx_v, rrows_v, gsem0, gsem1, osem0, osem1):
        idx = (idx0, idx1)
        rows = (rows0, rows1)
        gsem = (gsem0, gsem1)
        osem = (osem0, osem1)

        wid = lax.axis_index("s") * 2 + lax.axis_index("c")
        base = wid * rows_per_worker

        ridx_v[...] = jnp.minimum(lax.iota(jnp.int32, 16) + 1, 3)
        pltpu.async_copy(table_hbm.at[ridx_v], rrows_v, gsem0).wait()
        rep_lo = (rrows_v[0, pl.ds(0, 16)] + rrows_v[1, pl.ds(0, 16)]
                  + rrows_v[2, pl.ds(0, 16)])
        rep_hi = (rrows_v[0, pl.ds(16, 16)] + rrows_v[1, pl.ds(16, 16)]
                  + rrows_v[2, pl.ds(16, 16)])

        def start_gather(i, b):
            off = base + i * _CHUNK
            pltpu.sync_copy(x_hbm.at[pl.ds(off, _CHUNK)], idx[b])
            pltpu.async_copy(table_hbm.at[idx[b]], rows[b], gsem[b])

        def wait_gather(b):
            pltpu.make_async_copy(table_hbm.at[idx[b]], rows[b],
                                  gsem[b]).wait()

        def start_out(i, b):
            off = base + i * _CHUNK
            pltpu.async_copy(rows[b], out_hbm.at[pl.ds(off, _CHUNK)], osem[b])

        def wait_out(i, b):
            off = base + i * _CHUNK
            pltpu.make_async_copy(rows[b], out_hbm.at[pl.ds(off, _CHUNK)],
                                  osem[b]).wait()

        def fix_zero_rows(b):
            idx_b = idx[b]
            rows_b = rows[b]

            def vmin_body(g, mcarry):
                return jnp.minimum(mcarry, idx_b[pl.ds(g * 16, 16)])

            m = lax.fori_loop(1, _CHUNK // 16, vmin_body,
                              idx_b[pl.ds(0, 16)])
            s = m[0]
            for j in range(1, 16):
                s = jnp.minimum(s, m[j])

            @pl.when(s == 0)
            def _fix():
                def group_body(g, _):
                    iv = idx_b[pl.ds(g * 16, 16)]
                    for j in range(16):

                        @pl.when(iv[j] == 0)
                        def _wr(j=j):
                            row = g * 16 + j
                            rows_b[row, pl.ds(0, 16)] = rep_lo
                            rows_b[row, pl.ds(16, 16)] = rep_hi

                    return 0

                lax.fori_loop(0, _CHUNK // 16, group_body, 0)

        def finish_chunk(i, b):
            wait_gather(b)
            fix_zero_rows(b)
            start_out(i, b)

        start_gather(0, 0)
        start_gather(1, 1)
        finish_chunk(0, 0)

        def pair_body(i2, _):
            i_a = 2 + 2 * i2
            wait_out(i_a - 2, 0)
            start_gather(i_a, 0)
            finish_chunk(i_a - 1, 1)

            i_b = i_a + 1
            wait_out(i_b - 2, 1)
            start_gather(i_b, 1)
            finish_chunk(i_b - 1, 0)
            return 0

        lax.fori_loop(0, (num_chunks - 2) // 2, pair_body, 0)

        finish_chunk(num_chunks - 1, 1)
        wait_out(num_chunks - 2, 0)
        wait_out(num_chunks - 1, 1)

    return gather_kernel


def kernel(x, table):
    xf = x.reshape(-1).astype(jnp.int32)
    out = _make_gather(xf.shape[0])(xf, table)
    return out.reshape(x.shape + (_EMBED_DIM,))
